# baseline (device time: 62817 ns/iter reference)
import jax
import jax.numpy as jnp
from jax import lax
from jax.experimental import pallas as pl
from jax.experimental.pallas import tpu as pltpu

N_DEV = 4
SQ = 1024
SKV_LOC = 1024
HQ = 8
DH = 128
D = HQ * DH
NC = 8
CW = D // NC
LAG = 3
SCALE = 0.08838834764831843
BLK = 64


def kernel(x, Wq, K_ext, V_ext, Wo):
    k2 = K_ext.astype(jnp.bfloat16).reshape(SKV_LOC, D)
    v2 = V_ext.astype(jnp.bfloat16).reshape(SKV_LOC, D)

    def body(x_ref, wq_ref, k_ref, v_ref, wo_ref, out_ref,
             cself, cin1, csum, cin2,
             l_self, l_in1, l_sum, l_in2,
             s1_send, s1_recv, s2_send, s2_recv, l_send, l_recv):
        my = lax.axis_index("i")
        p1 = my ^ 1
        p2 = 3 - my

        barrier_sem = pltpu.get_barrier_semaphore()
        for nbr in (p1, p2):
            pl.semaphore_signal(
                barrier_sem, inc=1,
                device_id=(nbr,), device_id_type=pl.DeviceIdType.MESH,
            )

        xb = x_ref[0].astype(jnp.bfloat16)
        wqb = wq_ref[...].astype(jnp.bfloat16)
        q = jnp.dot(xb, wqb, preferred_element_type=jnp.float32)
        qb = (q * SCALE).astype(jnp.bfloat16)

        rows = lax.broadcasted_iota(jnp.int32, (SQ, SKV_LOC), 0) // BLK
        cols = (lax.broadcasted_iota(jnp.int32, (SQ, SKV_LOC), 1)
                + my * SKV_LOC) // BLK
        mask = (rows == cols) | (cols == 0) | (((rows + cols) % 3) == 0)

        def head_partial(h):
            sl = slice(h * DH, (h + 1) * DH)
            s = lax.dot_general(
                qb[:, sl], k_ref[:, sl], (((1,), (1,)), ((), ())),
                preferred_element_type=jnp.float32,
            )
            w = jnp.exp(jnp.where(mask, s, -1e9))
            ctx_h = jnp.dot(w.astype(jnp.bfloat16), v_ref[:, sl],
                            preferred_element_type=jnp.float32)
            cself[h] = ctx_h.astype(jnp.bfloat16)
            return jnp.sum(w, axis=1, keepdims=True)

        def partner_of(c, stage):
            return p1 if (c % 2 == 0) == (stage == 1) else p2

        def start_s1(c):
            rd = pltpu.make_async_remote_copy(
                src_ref=cself.at[c], dst_ref=cin1.at[c],
                send_sem=s1_send.at[c], recv_sem=s1_recv.at[c],
                device_id=(partner_of(c, 1),),
                device_id_type=pl.DeviceIdType.MESH,
            )
            rd.start()
            return rd

        def pairsum_start_s2(c, rd1):
            rd1.wait_recv()
            csum[c] = (cself[c].astype(jnp.float32)
                       + cin1[c].astype(jnp.float32)).astype(jnp.bfloat16)
            rd = pltpu.make_async_remote_copy(
                src_ref=csum.at[c], dst_ref=cin2.at[c],
                send_sem=s2_send.at[c], recv_sem=s2_recv.at[c],
                device_id=(partner_of(c, 2),),
                device_id_type=pl.DeviceIdType.MESH,
            )
            rd.start()
            return rd

        l_cols = []
        rd1 = [None] * NC
        rd2 = [None] * NC
        for c in range(NC):
            l_cols.append(head_partial(c))
            if c == 0:
                pl.semaphore_wait(barrier_sem, 2)
            rd1[c] = start_s1(c)
            if c >= LAG:
                rd2[c - LAG] = pairsum_start_s2(c - LAG, rd1[c - LAG])
        l_cols.append(jnp.zeros((SQ, 128 - HQ), jnp.float32))
        l_self[...] = jnp.concatenate(l_cols, axis=1).astype(jnp.bfloat16)
        rd_l1 = pltpu.make_async_remote_copy(
            src_ref=l_self, dst_ref=l_in1,
            send_sem=l_send.at[0], recv_sem=l_recv.at[0],
            device_id=(p1,), device_id_type=pl.DeviceIdType.MESH,
        )
        rd_l1.start()

        for c in range(NC - LAG, NC):
            rd2[c] = pairsum_start_s2(c, rd1[c])

        rd_l1.wait_recv()
        l_sum[...] = (l_self[...].astype(jnp.float32)
                      + l_in1[...].astype(jnp.float32)).astype(jnp.bfloat16)
        rd_l2 = pltpu.make_async_remote_copy(
            src_ref=l_sum, dst_ref=l_in2,
            send_sem=l_send.at[1], recv_sem=l_recv.at[1],
            device_id=(p2,), device_id_type=pl.DeviceIdType.MESH,
        )
        rd_l2.start()
        rd_l2.wait_recv()
        l_tot = (l_sum[...].astype(jnp.float32)
                 + l_in2[...].astype(jnp.float32))

        wob = wo_ref[...].astype(jnp.bfloat16)
        acc = None
        for c in range(NC):
            rd2[c].wait_recv()
            tot = (csum[c].astype(jnp.float32)
                   + cin2[c].astype(jnp.float32))
            norm_c = (tot / l_tot[:, c:c + 1]).astype(jnp.bfloat16)
            part = jnp.dot(norm_c, wob[c * CW:(c + 1) * CW, :],
                           preferred_element_type=jnp.float32)
            acc = part if acc is None else acc + part
        out_ref[0] = acc

        for rd in rd1 + rd2 + [rd_l1, rd_l2]:
            rd.wait_send()

    out = pl.pallas_call(
        body,
        out_shape=jax.ShapeDtypeStruct((1, SQ, D), jnp.float32),
        in_specs=[pl.BlockSpec(memory_space=pltpu.VMEM)] * 5,
        out_specs=pl.BlockSpec(memory_space=pltpu.VMEM),
        scratch_shapes=[
            pltpu.VMEM((NC, SQ, CW), jnp.bfloat16),
            pltpu.VMEM((NC, SQ, CW), jnp.bfloat16),
            pltpu.VMEM((NC, SQ, CW), jnp.bfloat16),
            pltpu.VMEM((NC, SQ, CW), jnp.bfloat16),
            pltpu.VMEM((SQ, 128), jnp.bfloat16),
            pltpu.VMEM((SQ, 128), jnp.bfloat16),
            pltpu.VMEM((SQ, 128), jnp.bfloat16),
            pltpu.VMEM((SQ, 128), jnp.bfloat16),
            pltpu.SemaphoreType.DMA((NC,)),
            pltpu.SemaphoreType.DMA((NC,)),
            pltpu.SemaphoreType.DMA((NC,)),
            pltpu.SemaphoreType.DMA((NC,)),
            pltpu.SemaphoreType.DMA((2,)),
            pltpu.SemaphoreType.DMA((2,)),
        ],
        compiler_params=pltpu.CompilerParams(collective_id=0),
    )(x, Wq, k2, v2, Wo)
    return out


# device time: 57608 ns/iter; 1.0904x vs baseline; 1.0904x over previous
import jax
import jax.numpy as jnp
from jax import lax
from jax.experimental import pallas as pl
from jax.experimental.pallas import tpu as pltpu

N_DEV = 4
SQ = 1024
SKV_LOC = 1024
HQ = 8
DH = 128
D = HQ * DH
NC = 4
CW = D // NC
SCALE = 0.08838834764831843
BLK = 64


def kernel(x, Wq, K_ext, V_ext, Wo):
    k2 = K_ext.astype(jnp.bfloat16).reshape(SKV_LOC, D)
    v2 = V_ext.astype(jnp.bfloat16).reshape(SKV_LOC, D)

    def body(x_ref, wq_ref, k_ref, v_ref, wo_ref, out_ref,
             cself, cin1, csum, cin2,
             l_self, l_in1, l_sum, l_in2,
             s1_send, s1_recv, s2_send, s2_recv, l_send, l_recv):
        my = lax.axis_index("i")
        p1 = my ^ 1
        p2 = 3 - my

        barrier_sem = pltpu.get_barrier_semaphore()
        for nbr in (p1, p2):
            pl.semaphore_signal(
                barrier_sem, inc=1,
                device_id=(nbr,), device_id_type=pl.DeviceIdType.MESH,
            )

        xb = x_ref[0].astype(jnp.bfloat16)
        wqb = (wq_ref[...] * SCALE).astype(jnp.bfloat16)
        qb = jnp.dot(xb, wqb,
                     preferred_element_type=jnp.float32
                     ).astype(jnp.bfloat16)

        rows = lax.broadcasted_iota(jnp.int32, (SQ, SKV_LOC), 0) // BLK
        cols = (lax.broadcasted_iota(jnp.int32, (SQ, SKV_LOC), 1)
                + my * SKV_LOC) // BLK
        mask = (rows == cols) | (cols == 0) | (((rows + cols) % 3) == 0)
        mask_bias = jnp.where(mask, 0.0, -1e9)

        def head_partial(h, c):
            sl = slice(h * DH, (h + 1) * DH)
            s = lax.dot_general(
                qb[:, sl], k_ref[:, sl], (((1,), (1,)), ((), ())),
                preferred_element_type=jnp.float32,
            )
            w = jnp.exp(s + mask_bias)
            ctx_h = jnp.dot(w.astype(jnp.bfloat16), v_ref[:, sl],
                            preferred_element_type=jnp.float32)
            off = (h % 2) * DH
            cself[c, :, off:off + DH] = ctx_h.astype(jnp.bfloat16)
            return jnp.sum(w, axis=1, keepdims=True)

        def partner_of(c, stage):
            return p1 if (c % 2 == 0) == (stage == 1) else p2

        def start_s1(c):
            rd = pltpu.make_async_remote_copy(
                src_ref=cself.at[c], dst_ref=cin1.at[c],
                send_sem=s1_send.at[c], recv_sem=s1_recv.at[c],
                device_id=(partner_of(c, 1),),
                device_id_type=pl.DeviceIdType.MESH,
            )
            rd.start()
            return rd

        def pairsum_start_s2(c, rd1):
            rd1.wait_recv()
            csum[c] = (cself[c].astype(jnp.float32)
                       + cin1[c].astype(jnp.float32)).astype(jnp.bfloat16)
            rd = pltpu.make_async_remote_copy(
                src_ref=csum.at[c], dst_ref=cin2.at[c],
                send_sem=s2_send.at[c], recv_sem=s2_recv.at[c],
                device_id=(partner_of(c, 2),),
                device_id_type=pl.DeviceIdType.MESH,
            )
            rd.start()
            return rd

        l_cols = []
        l_cols.append(head_partial(0, 0))
        l_cols.append(head_partial(1, 0))
        pl.semaphore_wait(barrier_sem, 2)
        rd1_0 = start_s1(0)

        l_cols.append(head_partial(2, 1))
        l_cols.append(head_partial(3, 1))
        rd1_1 = start_s1(1)
        l_cols.append(head_partial(4, 2))
        l_cols.append(head_partial(5, 2))
        rd1_2 = start_s1(2)
        rd2_0 = pairsum_start_s2(0, rd1_0)

        l_cols.append(head_partial(6, 3))
        l_cols.append(head_partial(7, 3))
        rd1_3 = start_s1(3)
        rd2_1 = pairsum_start_s2(1, rd1_1)

        l_cols.append(jnp.zeros((SQ, 128 - HQ), jnp.float32))
        l_self[...] = jnp.concatenate(l_cols, axis=1).astype(jnp.bfloat16)
        rd_l1 = pltpu.make_async_remote_copy(
            src_ref=l_self, dst_ref=l_in1,
            send_sem=l_send.at[0], recv_sem=l_recv.at[0],
            device_id=(p1,), device_id_type=pl.DeviceIdType.MESH,
        )
        rd_l1.start()

        rd2_2 = pairsum_start_s2(2, rd1_2)
        rd2_3 = pairsum_start_s2(3, rd1_3)

        rd_l1.wait_recv()
        l_sum[...] = (l_self[...].astype(jnp.float32)
                      + l_in1[...].astype(jnp.float32)).astype(jnp.bfloat16)
        rd_l2 = pltpu.make_async_remote_copy(
            src_ref=l_sum, dst_ref=l_in2,
            send_sem=l_send.at[1], recv_sem=l_recv.at[1],
            device_id=(p2,), device_id_type=pl.DeviceIdType.MESH,
        )
        rd_l2.start()
        rd_l2.wait_recv()
        l_tot = (l_sum[...].astype(jnp.float32)
                 + l_in2[...].astype(jnp.float32))

        wob = wo_ref[...].astype(jnp.bfloat16)
        acc = None
        for c, rd2 in enumerate((rd2_0, rd2_1, rd2_2, rd2_3)):
            rd2.wait_recv()
            tot = (csum[c].astype(jnp.float32)
                   + cin2[c].astype(jnp.float32))
            n_cols = []
            for j in range(2):
                h = 2 * c + j
                n_cols.append(tot[:, j * DH:(j + 1) * DH]
                              / l_tot[:, h:h + 1])
            norm_c = jnp.concatenate(n_cols, axis=1).astype(jnp.bfloat16)
            part = jnp.dot(norm_c, wob[c * CW:(c + 1) * CW, :],
                           preferred_element_type=jnp.float32)
            acc = part if acc is None else acc + part
        out_ref[0] = acc

        for rd in (rd1_0, rd1_1, rd1_2, rd1_3,
                   rd2_0, rd2_1, rd2_2, rd2_3, rd_l1, rd_l2):
            rd.wait_send()

    out = pl.pallas_call(
        body,
        out_shape=jax.ShapeDtypeStruct((1, SQ, D), jnp.float32),
        in_specs=[pl.BlockSpec(memory_space=pltpu.VMEM)] * 5,
        out_specs=pl.BlockSpec(memory_space=pltpu.VMEM),
        scratch_shapes=[
            pltpu.VMEM((NC, SQ, CW), jnp.bfloat16),
            pltpu.VMEM((NC, SQ, CW), jnp.bfloat16),
            pltpu.VMEM((NC, SQ, CW), jnp.bfloat16),
            pltpu.VMEM((NC, SQ, CW), jnp.bfloat16),
            pltpu.VMEM((SQ, 128), jnp.bfloat16),
            pltpu.VMEM((SQ, 128), jnp.bfloat16),
            pltpu.VMEM((SQ, 128), jnp.bfloat16),
            pltpu.VMEM((SQ, 128), jnp.bfloat16),
            pltpu.SemaphoreType.DMA((NC,)),
            pltpu.SemaphoreType.DMA((NC,)),
            pltpu.SemaphoreType.DMA((NC,)),
            pltpu.SemaphoreType.DMA((NC,)),
            pltpu.SemaphoreType.DMA((2,)),
            pltpu.SemaphoreType.DMA((2,)),
        ],
        compiler_params=pltpu.CompilerParams(collective_id=0),
    )(x, Wq, k2, v2, Wo)
    return out
